# Initial kernel scaffold; baseline (speedup 1.0000x reference)
import jax
import jax.numpy as jnp
from jax.experimental import pallas as pl


def kernel(distance_matrices, num_species_batch):
    raise NotImplementedError("write your pallas kernel here")


# speedup vs baseline: 13.2832x; 13.2832x over previous
import functools

import numpy as np
import jax
import jax.numpy as jnp
from jax import lax
from jax.experimental import pallas as pl
from jax.experimental.pallas import tpu as pltpu
from jax.experimental.pallas import tpu_sc as plsc

_RCA = 3.5
_SZ = np.array([0.19634954, 0.58904862, 0.9817477, 1.3744468,
                1.7671459, 2.1598449, 2.552544, 2.9452431], dtype=np.float64)
_SA = np.array([0.9, 1.55, 2.2, 2.85], dtype=np.float64)
_HC = np.tile(0.5 * np.cos(_SZ), 2).astype(np.float32)
_HS = np.tile(0.5 * np.sin(_SZ), 2).astype(np.float32)
_SA_LO = np.repeat(_SA[:2], 8).astype(np.float32)
_SA_HI = np.repeat(_SA[2:], 8).astype(np.float32)
_COSP = (0.9999999738948335, -0.49999985130227886, 0.04166646235582207,
         -0.0013887731795384876, 2.4769053365277362e-05,
         -2.7075450696039624e-07, 1.7243752160329109e-09)
_QCONST = np.concatenate([_HC, _HS, _SA_LO, _SA_HI]).astype(np.float32)

_B, _N, _NQ, _NP = 4, 48, 32, 10
_NW = 32
_APW = _B * _N // _NW
_WPB = _N // _APW
_ACC = _APW * _NP * _NQ


def _poly_cos(t):
    r = jnp.float32(_COSP[6])
    for c in _COSP[5::-1]:
        r = r * t + jnp.float32(c)
    return r


def _scal(ref, i):
    return ref[pl.ds(i, 16)][0]


def _aev_body(d_hbm, s_hbm, qc_hbm, out_hbm,
              d_v, s_v, qc_v, nd_v, ns_v, na_v, nfc_v, acc_v):
    wid = lax.axis_index("s") * 2 + lax.axis_index("c")
    b = wid // _WPB
    i0 = (wid % _WPB) * _APW
    pltpu.sync_copy(d_hbm.at[pl.ds(b * (_N * _N), _N * _N)],
                    d_v.at[pl.ds(0, _N * _N)])
    pltpu.sync_copy(s_hbm, s_v.at[pl.ds(0, _B * _N)])
    pltpu.sync_copy(qc_hbm, qc_v.at[pl.ds(0, 64)])
    sbase = b * _N

    zeros16 = jnp.zeros((16,), jnp.float32)

    def _zero(t, carry):
        acc_v[pl.ds(t * 16, 16)] = zeros16
        return carry
    lax.fori_loop(0, _ACC // 16, _zero, 0)

    hc = qc_v[pl.ds(0, 16)]
    hs = qc_v[pl.ds(16, 16)]
    sa_lo = qc_v[pl.ds(32, 16)]
    sa_hi = qc_v[pl.ds(48, 16)]
    pi_rca = jnp.float32(np.pi / _RCA)

    def _atom(ii, carry):
        iloc = i0 + ii
        svi = _scal(s_v, sbase + iloc)
        drow = iloc * _N

        n = jnp.int32(0)
        for c in range(3):
            dv = d_v[pl.ds(drow + c * 16, 16)]
            sv = s_v[pl.ds(sbase + c * 16, 16)]
            jv = lax.iota(jnp.int32, 16) + (c * 16)
            vi = jnp.full((16,), svi, jnp.int32)
            m = (dv < _RCA) & (jv != iloc) & (sv > 0) & (vi > 0)
            mi = m.astype(jnp.int32)
            pos = plsc.cumsum(mi) + (n - 1)
            plsc.store_scatter(nd_v, [pos], dv, mask=m)
            plsc.store_scatter(ns_v, [pos], sv, mask=m)
            plsc.store_scatter(na_v, [pos], jv, mask=m)
            n = n + jnp.sum(mi)

        for c in range(3):
            rv = nd_v[pl.ds(c * 16, 16)]
            u = rv * pi_rca
            nfc_v[pl.ds(c * 16, 16)] = 0.5 * _poly_cos(u * u) + 0.5

        acc0 = ii * (_NP * _NQ)

        def _jj(jj, cj):
            r_ij = _scal(nd_v, jj)
            sj = _scal(ns_v, jj)
            fcj = _scal(nfc_v, jj)
            ja = _scal(na_v, jj)
            rij2 = r_ij * r_ij
            two_rij = r_ij + r_ij
            fcj2 = fcj + fcj
            jrow = ja * _N

            def _kk(kk, ck):
                r_ik = _scal(nd_v, kk)
                sk = _scal(ns_v, kk)
                fck = _scal(nfc_v, kk)
                ka = _scal(na_v, kk)
                r_jk = _scal(d_v, jrow + ka)
                num = rij2 + r_ik * r_ik - r_jk * r_jk
                den = jnp.maximum(two_rij * r_ik, jnp.float32(1e-10))
                cav = jnp.full((16,), num, jnp.float32) / jnp.full(
                    (16,), den, jnp.float32)
                cv = jnp.float32(0.95) * cav
                t1 = jnp.maximum(jnp.float32(1.0) - cv * cv, jnp.float32(1e-20))
                ti = plsc.bitcast(t1, jnp.int32)
                yi = jnp.int32(0x5F3759DF) - lax.shift_right_logical(ti, 1)
                y = plsc.bitcast(yi, jnp.float32)
                for _ in range(3):
                    y = y * (jnp.float32(1.5) - jnp.float32(0.5) * t1 * y * y)
                sinv = t1 * y
                x = jnp.float32(0.5) + cv * hc + sinv * hs
                f1 = x * x
                f1 = f1 * f1
                f1 = f1 * f1
                f1 = f1 * f1
                f1 = f1 * f1
                ravg = jnp.float32(0.5) * (r_ij + r_ik)
                w = fcj2 * fck
                e1 = ravg - sa_lo
                e2 = ravg - sa_hi
                f2a = jnp.exp(jnp.float32(-8.0) * (e1 * e1))
                f2b = jnp.exp(jnp.float32(-8.0) * (e2 * e2))
                wf1 = w * f1
                amin = jnp.minimum(sj, sk) - 1
                amax = jnp.maximum(sj, sk) - 1
                p = (amin * 4 - lax.shift_right_arithmetic(amin * (amin - 1), 1)
                     + (amax - amin))
                off = acc0 + p * _NQ
                plsc.addupdate(acc_v.at[pl.ds(off, 16)], wf1 * f2a)
                plsc.addupdate(acc_v.at[pl.ds(off + 16, 16)], wf1 * f2b)
                return ck

            return lax.fori_loop(jj + 1, n, _kk, cj)

        lax.fori_loop(0, n, _jj, 0)
        return carry

    lax.fori_loop(0, _APW, _atom, 0)
    pltpu.sync_copy(acc_v, out_hbm.at[pl.ds(wid * _ACC, _ACC)])


_aev_sc = functools.partial(
    pl.kernel,
    out_type=jax.ShapeDtypeStruct((_B * _N * _NP * _NQ,), jnp.float32),
    mesh=plsc.VectorSubcoreMesh(core_axis_name="c", subcore_axis_name="s"),
    scratch_types=[
        pltpu.VMEM((_N * _N + 16,), jnp.float32),
        pltpu.VMEM((_B * _N + 16,), jnp.int32),
        pltpu.VMEM((64,), jnp.float32),
        pltpu.VMEM((64,), jnp.float32),
        pltpu.VMEM((64,), jnp.int32),
        pltpu.VMEM((64,), jnp.int32),
        pltpu.VMEM((64,), jnp.float32),
        pltpu.VMEM((_ACC,), jnp.float32),
    ],
    compiler_params=pltpu.CompilerParams(needs_layout_passes=False),
)(_aev_body)


def kernel(distance_matrices, num_species_batch):
    out = _aev_sc(distance_matrices.reshape(_B * _N * _N),
                  num_species_batch.reshape(_B * _N), jnp.asarray(_QCONST))
    return out.reshape(_B, _N, _NP * _NQ)


# speedup vs baseline: 18.4264x; 1.3872x over previous
import functools

import numpy as np
import jax
import jax.numpy as jnp
from jax import lax
from jax.experimental import pallas as pl
from jax.experimental.pallas import tpu as pltpu
from jax.experimental.pallas import tpu_sc as plsc

_RCA = 3.5
_SZ = np.array([0.19634954, 0.58904862, 0.9817477, 1.3744468,
                1.7671459, 2.1598449, 2.552544, 2.9452431], dtype=np.float64)
_SA = np.array([0.9, 1.55, 2.2, 2.85], dtype=np.float64)
_HC = (0.5 * np.cos(_SZ)).astype(np.float32)
_HS = (0.5 * np.sin(_SZ)).astype(np.float32)
_SAF = _SA.astype(np.float32)
_COSP = (0.9999999738948335, -0.49999985130227886, 0.04166646235582207,
         -0.0013887731795384876, 2.4769053365277362e-05,
         -2.7075450696039624e-07, 1.7243752160329109e-09)

_B, _N, _NQ, _NP = 4, 48, 32, 10
_NW = 32
_APW = _B * _N // _NW
_WPB = _N // _APW
_ACC = _APW * _NP * _NQ
_MAXPAIR = (_N - 1) * (_N - 2) // 2 + 32


def _poly_cos(t):
    r = jnp.float32(_COSP[6])
    for c in _COSP[5::-1]:
        r = r * t + jnp.float32(c)
    return r


def _scal(ref, i):
    return ref[pl.ds(i, 16)][0]


def _aev_body(d_hbm, s_hbm, out_hbm,
              d_v, s_v, nd_v, ns_v, na_v, nfc_v, pj_v, pk_v, acc_v):
    wid = lax.axis_index("s") * 2 + lax.axis_index("c")
    b = wid // _WPB
    i0 = (wid % _WPB) * _APW
    pltpu.sync_copy(d_hbm.at[pl.ds(b * (_N * _N), _N * _N)],
                    d_v.at[pl.ds(0, _N * _N)])
    pltpu.sync_copy(s_hbm, s_v.at[pl.ds(0, _B * _N)])
    sbase = b * _N

    zeros16 = jnp.zeros((16,), jnp.float32)
    izeros16 = jnp.zeros((16,), jnp.int32)

    def _zero(t, carry):
        acc_v[pl.ds(t * 16, 16)] = zeros16
        return carry
    lax.fori_loop(0, _ACC // 16, _zero, 0)

    pi_rca = jnp.float32(np.pi / _RCA)

    def _atom(ii, carry):
        iloc = i0 + ii
        svi = _scal(s_v, sbase + iloc)
        drow = iloc * _N

        n = jnp.int32(0)
        for c in range(3):
            dv = d_v[pl.ds(drow + c * 16, 16)]
            sv = s_v[pl.ds(sbase + c * 16, 16)]
            jv = lax.iota(jnp.int32, 16) + (c * 16)
            vi = jnp.full((16,), svi, jnp.int32)
            m = (dv < _RCA) & (jv != iloc) & (sv > 0) & (vi > 0)
            mi = m.astype(jnp.int32)
            pos = plsc.cumsum(mi) + (n - 1)
            plsc.store_scatter(nd_v, [pos], dv, mask=m)
            plsc.store_scatter(ns_v, [pos], sv, mask=m)
            plsc.store_scatter(na_v, [pos], jv, mask=m)
            n = n + jnp.sum(mi)

        for c in range(3):
            rv = nd_v[pl.ds(c * 16, 16)]
            u = rv * pi_rca
            nfc_v[pl.ds(c * 16, 16)] = 0.5 * _poly_cos(u * u) + 0.5

        def _enum(jj, cnt):
            rem = n - 1 - jj

            def _ec(c, carry2):
                kv = lax.iota(jnp.int32, 16) + (jj + 1 + c * 16)
                pj_v[pl.ds(cnt + c * 16, 16)] = jnp.full((16,), jj, jnp.int32)
                pk_v[pl.ds(cnt + c * 16, 16)] = kv
                return carry2

            lax.fori_loop(0, lax.shift_right_logical(rem + 15, 4), _ec, 0)
            return cnt + rem

        tcount = lax.fori_loop(0, n, _enum, jnp.int32(0))
        pj_v[pl.ds(tcount, 16)] = izeros16
        pk_v[pl.ds(tcount, 16)] = izeros16

        acc0 = ii * (_NP * _NQ)

        def _chunk(c2, carry2):
            base = c2 * 16
            live = lax.iota(jnp.int32, 16) + base < tcount
            pj = pj_v[pl.ds(base, 16)]
            pk = pk_v[pl.ds(base, 16)]
            r_ij = plsc.load_gather(nd_v, [pj])
            r_ik = plsc.load_gather(nd_v, [pk])
            fcj = plsc.load_gather(nfc_v, [pj])
            fck = plsc.load_gather(nfc_v, [pk])
            sj = plsc.load_gather(ns_v, [pj])
            sk = plsc.load_gather(ns_v, [pk])
            ja = plsc.load_gather(na_v, [pj])
            ka = plsc.load_gather(na_v, [pk])
            r_jk = plsc.load_gather(d_v, [ja * _N + ka])
            num = r_ij * r_ij + r_ik * r_ik - r_jk * r_jk
            den = jnp.maximum((r_ij + r_ij) * r_ik, jnp.float32(1e-10))
            cv = jnp.float32(0.95) * (num / den)
            t1 = jnp.maximum(jnp.float32(1.0) - cv * cv, jnp.float32(1e-20))
            ti = plsc.bitcast(t1, jnp.int32)
            yi = jnp.int32(0x5F3759DF) - lax.shift_right_logical(ti, 1)
            y = plsc.bitcast(yi, jnp.float32)
            for _ in range(3):
                y = y * (jnp.float32(1.5) - jnp.float32(0.5) * t1 * y * y)
            sinv = t1 * y
            ravg = jnp.float32(0.5) * (r_ij + r_ik)
            w = (fcj + fcj) * fck
            amin = jnp.minimum(sj, sk) - 1
            amax = jnp.maximum(sj, sk) - 1
            p = (amin * 4 - lax.shift_right_arithmetic(amin * (amin - 1), 1)
                 + (amax - amin))
            pofs = p * _NQ + acc0
            wf2 = []
            for a in range(4):
                e = ravg - jnp.float32(_SAF[a])
                wf2.append(w * jnp.exp(jnp.float32(-8.0) * (e * e)))
            for z in range(8):
                x = (jnp.float32(0.5) + cv * jnp.float32(_HC[z])
                     + sinv * jnp.float32(_HS[z]))
                f1 = x * x
                f1 = f1 * f1
                f1 = f1 * f1
                f1 = f1 * f1
                f1 = f1 * f1
                for a in range(4):
                    plsc.addupdate_scatter(acc_v, [pofs + (a * 8 + z)],
                                           f1 * wf2[a], mask=live)
            return carry2

        lax.fori_loop(0, lax.shift_right_logical(tcount + 15, 4), _chunk, 0)
        return carry

    lax.fori_loop(0, _APW, _atom, 0)
    pltpu.sync_copy(acc_v, out_hbm.at[pl.ds(wid * _ACC, _ACC)])


_aev_sc = functools.partial(
    pl.kernel,
    out_type=jax.ShapeDtypeStruct((_B * _N * _NP * _NQ,), jnp.float32),
    mesh=plsc.VectorSubcoreMesh(core_axis_name="c", subcore_axis_name="s"),
    scratch_types=[
        pltpu.VMEM((_N * _N + 16,), jnp.float32),
        pltpu.VMEM((_B * _N + 16,), jnp.int32),
        pltpu.VMEM((64,), jnp.float32),
        pltpu.VMEM((64,), jnp.int32),
        pltpu.VMEM((64,), jnp.int32),
        pltpu.VMEM((64,), jnp.float32),
        pltpu.VMEM((_MAXPAIR,), jnp.int32),
        pltpu.VMEM((_MAXPAIR,), jnp.int32),
        pltpu.VMEM((_ACC,), jnp.float32),
    ],
    compiler_params=pltpu.CompilerParams(needs_layout_passes=False),
)(_aev_body)


def kernel(distance_matrices, num_species_batch):
    out = _aev_sc(distance_matrices.reshape(_B * _N * _N),
                  num_species_batch.reshape(_B * _N))
    return out.reshape(_B, _N, _NP * _NQ)
